# Initial kernel scaffold; baseline (speedup 1.0000x reference)
import jax
import jax.numpy as jnp
from jax.experimental import pallas as pl


def kernel(x, edge_index, batch, W1_l, W1_r, b1, W2_l, W2_r, b2):
    raise NotImplementedError("write your pallas kernel here")


# speedup vs baseline: 4.2484x; 4.2484x over previous
import functools

import jax
import jax.numpy as jnp
from jax import lax
from jax.experimental import pallas as pl
from jax.experimental.pallas import tpu as pltpu
import jax.experimental.pallas.tpu_sc as plsc

N = 10000
E = 320000
D = 128
G = 16

NSC = 2
NSUB = 16
NW = NSC * NSUB
EPW = E // NW
CHUNK = 128
SLAB = 4
NC = 80
NSLAB = NC // SLAB
NSLAB_A = NSLAB + 2


DUMMY = N
N_PAD = 10112
RPT = N_PAD // NSUB

BR = 1000
GRID = N // BR

_mesh = plsc.VectorSubcoreMesh(core_axis_name="c", subcore_axis_name="s")


@functools.partial(
    pl.kernel,
    out_type=jax.ShapeDtypeStruct((NSC, N_PAD, D), jnp.float32),
    mesh=_mesh,
    scratch_types=[
        pltpu.VMEM((NC, CHUNK), jnp.int32),
        pltpu.VMEM((CHUNK, D), jnp.float32),
        pltpu.VMEM_SHARED((N_PAD, D), jnp.float32),
    ],
)
def _sc_hist(idx_hbm, ones_hbm, zer_hbm, out, sv, ones_v, acc):
    c = lax.axis_index("c")
    s = lax.axis_index("s")
    wid = c * NSUB + s
    pltpu.sync_copy(idx_hbm.at[wid], sv)
    pltpu.sync_copy(ones_hbm, ones_v)
    pltpu.sync_copy(zer_hbm, acc.at[pl.ds(s * RPT, RPT)])
    plsc.subcore_barrier()

    def body(i, carry):
        pltpu.sync_copy(ones_v, acc.at[sv.at[i]], add=True)
        return carry

    lax.fori_loop(0, NC, body, 0)
    plsc.subcore_barrier()
    pltpu.sync_copy(acc.at[pl.ds(s * RPT, RPT)], out.at[c, pl.ds(s * RPT, RPT)])


@functools.partial(
    pl.kernel,
    out_type=jax.ShapeDtypeStruct((NSC, N_PAD, D), jnp.float32),
    mesh=_mesh,
    scratch_types=[
        pltpu.VMEM((SLAB, CHUNK), jnp.int32),
        pltpu.VMEM((SLAB, CHUNK), jnp.int32),
        pltpu.VMEM((SLAB, CHUNK), jnp.int32),
        pltpu.VMEM((SLAB, CHUNK), jnp.int32),
        pltpu.VMEM((CHUNK, D), jnp.float32),
        pltpu.VMEM((CHUNK, D), jnp.float32),
        pltpu.VMEM_SHARED((N_PAD, D), jnp.float32),
        pltpu.SemaphoreType.DMA,
        pltpu.SemaphoreType.DMA,
        pltpu.SemaphoreType.DMA,
        pltpu.SemaphoreType.DMA,
    ],
)
def _sc_edge_pass(table, gidx_hbm, sidx_hbm, zer_hbm, out,
                  gA, sA, gB, sB, buf0, buf1, acc, sem0, sem1, isemA, isemB):
    c = lax.axis_index("c")
    s = lax.axis_index("s")
    wid = c * NSUB + s
    gh = gidx_hbm.at[wid]
    sh = sidx_hbm.at[wid]
    rb = (buf0, buf1)
    sem = (sem0, sem1)

    pltpu.sync_copy(gh.at[0], gA)
    pltpu.sync_copy(sh.at[0], sA)
    pltpu.async_copy(gh.at[1], gB, isemB)
    pltpu.async_copy(sh.at[1], sB, isemB)
    pltpu.sync_copy(zer_hbm, acc.at[pl.ds(s * RPT, RPT)])
    plsc.subcore_barrier()

    pltpu.async_copy(table.at[gA.at[0]], buf0, sem0)
    pltpu.async_copy(table.at[gA.at[1]], buf1, sem1)

    def _half(i, g_cur, s_cur, g_nxt, s_nxt, isem_nxt):
        for k in range(SLAB):
            b = k % 2
            pltpu.make_async_copy(table.at[g_cur.at[k]], rb[b], sem[b]).wait()
            pltpu.sync_copy(rb[b], acc.at[s_cur.at[k]], add=True)
            if k == SLAB - 3:
                pltpu.make_async_copy(gh.at[i], g_nxt, isem_nxt).wait()
                pltpu.make_async_copy(sh.at[i], s_nxt, isem_nxt).wait()
            if k < SLAB - 2:
                pltpu.async_copy(table.at[g_cur.at[k + 2]], rb[b], sem[b])
            else:
                pltpu.async_copy(table.at[g_nxt.at[k - (SLAB - 2)]], rb[b], sem[b])

    def body(i, carry):
        jA = 2 * i
        _half(jA + 1, gA, sA, gB, sB, isemB)
        pltpu.async_copy(gh.at[jA + 2], gA, isemA)
        pltpu.async_copy(sh.at[jA + 2], sA, isemA)
        _half(jA + 2, gB, sB, gA, sA, isemA)
        pltpu.async_copy(gh.at[jA + 3], gB, isemB)
        pltpu.async_copy(sh.at[jA + 3], sB, isemB)
        return carry

    lax.fori_loop(0, NSLAB // 2, body, 0)
    pltpu.make_async_copy(gh.at[NSLAB + 1], gB, isemB).wait()
    pltpu.make_async_copy(sh.at[NSLAB + 1], sB, isemB).wait()
    pltpu.make_async_copy(table.at[gA.at[0]], buf0, sem0).wait()
    pltpu.make_async_copy(table.at[gA.at[1]], buf1, sem1).wait()
    plsc.subcore_barrier()
    pltpu.sync_copy(acc.at[pl.ds(s * RPT, RPT)], out.at[c, pl.ds(s * RPT, RPT)])


def _tc_xd_body(h_ref, x_ref, o_ref):
    deg = h_ref[0][:, 0:1] + h_ref[1][:, 0:1]
    dis = jnp.where(deg > 0.0, lax.rsqrt(deg), 0.0)
    o_ref[...] = x_ref[...] * dis


def _tc_dense1_body(a_ref, h_ref, x_ref, wlx, wll, wrx, wrl, y_ref, r_ref):
    deg = h_ref[0][:, 0:1] + h_ref[1][:, 0:1]
    dis = jnp.where(deg > 0.0, lax.rsqrt(deg), 0.0)
    xb = x_ref[...]
    lap = xb - dis * (a_ref[0] + a_ref[1])
    y_ref[...] = (jnp.dot(xb, wlx[...], preferred_element_type=jnp.float32)
                  + jnp.dot(lap, wll[...], preferred_element_type=jnp.float32))
    r_ref[...] = (jnp.dot(xb, wrx[...], preferred_element_type=jnp.float32)
                  + jnp.dot(lap, wrl[...], preferred_element_type=jnp.float32))


def _tc_dense2_body(a_ref, h_ref, r_ref, b_ref, wl, wr, y_ref, r2_ref):
    cnt = jnp.maximum(h_ref[0][:, 0:1] + h_ref[1][:, 0:1], 1.0)
    pre = (a_ref[0] + a_ref[1]) / cnt + b_ref[...] + r_ref[...]
    h = 0.5 * pre * (1.0 + lax.erf(pre * 0.7071067811865476))
    y_ref[...] = jnp.dot(h, wl[...], preferred_element_type=jnp.float32)
    r2_ref[...] = jnp.dot(h, wr[...], preferred_element_type=jnp.float32)


def _tc_pool_body(a_ref, h_ref, r_ref, b_ref, batch_ref, o_ref, pacc, cacc):
    i = pl.program_id(0)
    cnt = jnp.maximum(h_ref[0][:, 0:1] + h_ref[1][:, 0:1], 1.0)
    h2 = (a_ref[0] + a_ref[1]) / cnt + b_ref[...] + r_ref[...]
    bvec = batch_ref[0]
    gids = lax.broadcasted_iota(jnp.int32, (G, BR), 0)
    onehot = (bvec == gids).astype(jnp.float32)
    p = jnp.dot(onehot, h2, preferred_element_type=jnp.float32)
    cn = jnp.sum(onehot, axis=1, keepdims=True)

    @pl.when(i == 0)
    def _():
        pacc[...] = p
        cacc[...] = jnp.zeros_like(cacc) + cn

    @pl.when(i > 0)
    def _():
        pacc[...] += p
        cacc[...] += cn

    @pl.when(i == GRID - 1)
    def _():
        o_ref[...] = pacc[...] / jnp.maximum(cacc[...], 1.0)


def _pack_hist(idx, pad_val):
    a = idx.reshape(NW, EPW)
    a = jnp.pad(a, ((0, 0), (0, NC * CHUNK - EPW)), constant_values=pad_val)
    return a.reshape(NW, NC, CHUNK)


def _pack_edge(idx, pad_val):
    a = idx.reshape(NW, EPW)
    a = jnp.pad(a, ((0, 0), (0, NSLAB_A * SLAB * CHUNK - EPW)),
                constant_values=pad_val)
    return a.reshape(NW, NSLAB_A, SLAB, CHUNK)


def kernel(x, edge_index, batch, W1_l, W1_r, b1, W2_l, W2_r, b2):
    f32 = jnp.float32
    src = edge_index[0].astype(jnp.int32)
    dst = edge_index[1].astype(jnp.int32)

    hs_src = _pack_hist(src, DUMMY)
    hs_dst = _pack_hist(dst, DUMMY)
    g_dst = _pack_edge(dst, 0)
    s_src = _pack_edge(src, DUMMY)
    g_src = _pack_edge(src, 0)
    s_dst = _pack_edge(dst, DUMMY)

    onesD = jnp.ones((CHUNK, D), f32)
    zerD = jnp.zeros((RPT, D), f32)

    hsrc = _sc_hist(hs_src, onesD, zerD)
    hdst = _sc_hist(hs_dst, onesD, zerD)

    a_spec = pl.BlockSpec((NSC, BR, D), lambda i: (0, i, 0))
    h_spec = a_spec
    n_spec = pl.BlockSpec((BR, D), lambda i: (i, 0))
    w_spec = pl.BlockSpec((D, D), lambda i: (0, 0))
    b_spec = pl.BlockSpec((1, D), lambda i: (0, 0))
    nd_struct = jax.ShapeDtypeStruct((N, D), f32)

    xd = pl.pallas_call(
        _tc_xd_body,
        grid=(GRID,),
        in_specs=[h_spec, n_spec],
        out_specs=n_spec,
        out_shape=nd_struct,
    )(hsrc, x)

    accA = _sc_edge_pass(xd, g_dst, s_src, zerD)

    y1, r1 = pl.pallas_call(
        _tc_dense1_body,
        grid=(GRID,),
        in_specs=[a_spec, h_spec, n_spec, w_spec, w_spec, w_spec, w_spec],
        out_specs=[n_spec, n_spec],
        out_shape=[nd_struct, nd_struct],
    )(accA, hsrc, x, W1_l[:D], W1_l[D:], W1_r[:D], W1_r[D:])

    accB = _sc_edge_pass(y1, g_src, s_dst, zerD)

    y2, r2 = pl.pallas_call(
        _tc_dense2_body,
        grid=(GRID,),
        in_specs=[a_spec, h_spec, n_spec, b_spec, w_spec, w_spec],
        out_specs=[n_spec, n_spec],
        out_shape=[nd_struct, nd_struct],
    )(accB, hdst, r1, b1.reshape(1, D), W2_l, W2_r)

    accC = _sc_edge_pass(y2, g_src, s_dst, zerD)

    batch3 = batch.astype(jnp.int32).reshape(GRID, 1, BR)
    out = pl.pallas_call(
        _tc_pool_body,
        grid=(GRID,),
        in_specs=[a_spec, h_spec, n_spec, b_spec,
                  pl.BlockSpec((1, 1, BR), lambda i: (i, 0, 0))],
        out_specs=pl.BlockSpec((G, D), lambda i: (0, 0)),
        out_shape=jax.ShapeDtypeStruct((G, D), f32),
        scratch_shapes=[pltpu.VMEM((G, D), f32), pltpu.VMEM((G, 1), f32)],
    )(accC, hdst, r2, b2.reshape(1, D), batch3)
    return out


# speedup vs baseline: 5.0512x; 1.1890x over previous
import functools

import jax
import jax.numpy as jnp
from jax import lax
from jax.experimental import pallas as pl
from jax.experimental.pallas import tpu as pltpu
import jax.experimental.pallas.tpu_sc as plsc

N = 10000
E = 320000
D = 128
G = 16

NSC = 2
NSUB = 16
NW = NSC * NSUB
EPW = E // NW
CHUNK = 128
NC = 80

CH2 = 32
R_BUF = 5
UNROLL = 20
TRIPS = 16
NROW = 88


DUMMY = N
N_PAD = 10112
RPT = N_PAD // NSUB

BR = 1000
GRID = N // BR

_mesh = plsc.VectorSubcoreMesh(core_axis_name="c", subcore_axis_name="s")


@functools.partial(
    pl.kernel,
    out_type=jax.ShapeDtypeStruct((NSC, N_PAD, D), jnp.float32),
    mesh=_mesh,
    scratch_types=[
        pltpu.VMEM((NC, CHUNK), jnp.int32),
        pltpu.VMEM((CHUNK, D), jnp.float32),
        pltpu.VMEM_SHARED((N_PAD, D), jnp.float32),
    ],
)
def _sc_hist(idx_hbm, ones_hbm, zer_hbm, out, sv, ones_v, acc):
    c = lax.axis_index("c")
    s = lax.axis_index("s")
    wid = c * NSUB + s
    pltpu.sync_copy(idx_hbm.at[wid], sv)
    pltpu.sync_copy(ones_hbm, ones_v)
    pltpu.sync_copy(zer_hbm, acc.at[pl.ds(s * RPT, RPT)])
    plsc.subcore_barrier()

    def body(i, carry):
        pltpu.sync_copy(ones_v, acc.at[sv.at[i]], add=True)
        return carry

    lax.fori_loop(0, NC, body, 0)
    plsc.subcore_barrier()
    pltpu.sync_copy(acc.at[pl.ds(s * RPT, RPT)], out.at[c, pl.ds(s * RPT, RPT)])


@functools.partial(
    pl.kernel,
    out_type=jax.ShapeDtypeStruct((NSC, N_PAD, D), jnp.float32),
    mesh=_mesh,
    scratch_types=[
        pltpu.VMEM((NROW, 128), jnp.int32),
        pltpu.VMEM((NROW, 128), jnp.int32),
        pltpu.VMEM((CH2, D), jnp.float32),
        pltpu.VMEM((CH2, D), jnp.float32),
        pltpu.VMEM((CH2, D), jnp.float32),
        pltpu.VMEM((CH2, D), jnp.float32),
        pltpu.VMEM((CH2, D), jnp.float32),
        pltpu.VMEM_SHARED((N_PAD, D), jnp.float32),
        pltpu.SemaphoreType.DMA,
        pltpu.SemaphoreType.DMA,
        pltpu.SemaphoreType.DMA,
        pltpu.SemaphoreType.DMA,
        pltpu.SemaphoreType.DMA,
    ],
)
def _sc_edge_pass(table, gidx_hbm, sidx_hbm, zer_hbm, out,
                  gv, sv, b0, b1, b2, b3, b4, acc, s0, s1, s2, s3, s4):
    c = lax.axis_index("c")
    s = lax.axis_index("s")
    wid = c * NSUB + s
    bufs = (b0, b1, b2, b3, b4)
    sems = (s0, s1, s2, s3, s4)

    pltpu.sync_copy(gidx_hbm.at[wid], gv)
    pltpu.sync_copy(sidx_hbm.at[wid], sv)
    pltpu.sync_copy(zer_hbm, acc.at[pl.ds(s * RPT, RPT)])
    plsc.subcore_barrier()

    for j in range(R_BUF):
        pltpu.async_copy(table.at[gv.at[j // 4, pl.ds((j % 4) * CH2, CH2)]],
                         bufs[j], sems[j])

    def body(i, carry):
        base = i * (UNROLL // 4)
        for j in range(UNROLL):
            b = j % R_BUF
            pltpu.make_async_copy(
                table.at[gv.at[base + j // 4, pl.ds((j % 4) * CH2, CH2)]],
                bufs[b], sems[b]).wait()
            pltpu.sync_copy(
                bufs[b],
                acc.at[sv.at[base + j // 4, pl.ds((j % 4) * CH2, CH2)]],
                add=True)
            jn = j + R_BUF
            pltpu.async_copy(
                table.at[gv.at[base + jn // 4, pl.ds((jn % 4) * CH2, CH2)]],
                bufs[b], sems[b])
        return carry

    lax.fori_loop(0, TRIPS, body, 0)
    base = TRIPS * (UNROLL // 4)
    for j in range(R_BUF):
        pltpu.make_async_copy(
            table.at[gv.at[base + j // 4, pl.ds((j % 4) * CH2, CH2)]],
            bufs[j], sems[j]).wait()
    plsc.subcore_barrier()
    pltpu.sync_copy(acc.at[pl.ds(s * RPT, RPT)], out.at[c, pl.ds(s * RPT, RPT)])


def _tc_xd_body(h_ref, x_ref, o_ref):
    deg = h_ref[0][:, 0:1] + h_ref[1][:, 0:1]
    dis = jnp.where(deg > 0.0, lax.rsqrt(deg), 0.0)
    o_ref[...] = x_ref[...] * dis


def _tc_dense1_body(a_ref, h_ref, x_ref, wlx, wll, wrx, wrl, y_ref, r_ref):
    deg = h_ref[0][:, 0:1] + h_ref[1][:, 0:1]
    dis = jnp.where(deg > 0.0, lax.rsqrt(deg), 0.0)
    xb = x_ref[...]
    lap = xb - dis * (a_ref[0] + a_ref[1])
    y_ref[...] = (jnp.dot(xb, wlx[...], preferred_element_type=jnp.float32)
                  + jnp.dot(lap, wll[...], preferred_element_type=jnp.float32))
    r_ref[...] = (jnp.dot(xb, wrx[...], preferred_element_type=jnp.float32)
                  + jnp.dot(lap, wrl[...], preferred_element_type=jnp.float32))


def _tc_dense2_body(a_ref, h_ref, r_ref, b_ref, wl, wr, y_ref, r2_ref):
    cnt = jnp.maximum(h_ref[0][:, 0:1] + h_ref[1][:, 0:1], 1.0)
    pre = (a_ref[0] + a_ref[1]) / cnt + b_ref[...] + r_ref[...]
    h = 0.5 * pre * (1.0 + lax.erf(pre * 0.7071067811865476))
    y_ref[...] = jnp.dot(h, wl[...], preferred_element_type=jnp.float32)
    r2_ref[...] = jnp.dot(h, wr[...], preferred_element_type=jnp.float32)


def _tc_pool_body(a_ref, h_ref, r_ref, b_ref, batch_ref, o_ref, pacc, cacc):
    i = pl.program_id(0)
    cnt = jnp.maximum(h_ref[0][:, 0:1] + h_ref[1][:, 0:1], 1.0)
    h2 = (a_ref[0] + a_ref[1]) / cnt + b_ref[...] + r_ref[...]
    bvec = batch_ref[0]
    gids = lax.broadcasted_iota(jnp.int32, (G, BR), 0)
    onehot = (bvec == gids).astype(jnp.float32)
    p = jnp.dot(onehot, h2, preferred_element_type=jnp.float32)
    cn = jnp.sum(onehot, axis=1, keepdims=True)

    @pl.when(i == 0)
    def _():
        pacc[...] = p
        cacc[...] = jnp.zeros_like(cacc) + cn

    @pl.when(i > 0)
    def _():
        pacc[...] += p
        cacc[...] += cn

    @pl.when(i == GRID - 1)
    def _():
        o_ref[...] = pacc[...] / jnp.maximum(cacc[...], 1.0)


def _pack_hist(idx, pad_val):
    a = idx.reshape(NW, EPW)
    a = jnp.pad(a, ((0, 0), (0, NC * CHUNK - EPW)), constant_values=pad_val)
    return a.reshape(NW, NC, CHUNK)


def _pack_edge(idx, pad_val):
    a = idx.reshape(NW, EPW)
    a = jnp.pad(a, ((0, 0), (0, NROW * 128 - EPW)), constant_values=pad_val)
    return a.reshape(NW, NROW, 128)


def kernel(x, edge_index, batch, W1_l, W1_r, b1, W2_l, W2_r, b2):
    f32 = jnp.float32
    src = edge_index[0].astype(jnp.int32)
    dst = edge_index[1].astype(jnp.int32)

    hs_src = _pack_hist(src, DUMMY)
    hs_dst = _pack_hist(dst, DUMMY)
    g_dst = _pack_edge(dst, 0)
    s_src = _pack_edge(src, DUMMY)
    g_src = _pack_edge(src, 0)
    s_dst = _pack_edge(dst, DUMMY)

    onesD = jnp.ones((CHUNK, D), f32)
    zerD = jnp.zeros((RPT, D), f32)

    hsrc = _sc_hist(hs_src, onesD, zerD)
    hdst = _sc_hist(hs_dst, onesD, zerD)

    a_spec = pl.BlockSpec((NSC, BR, D), lambda i: (0, i, 0))
    h_spec = a_spec
    n_spec = pl.BlockSpec((BR, D), lambda i: (i, 0))
    w_spec = pl.BlockSpec((D, D), lambda i: (0, 0))
    b_spec = pl.BlockSpec((1, D), lambda i: (0, 0))
    nd_struct = jax.ShapeDtypeStruct((N, D), f32)

    xd = pl.pallas_call(
        _tc_xd_body,
        grid=(GRID,),
        in_specs=[h_spec, n_spec],
        out_specs=n_spec,
        out_shape=nd_struct,
    )(hsrc, x)

    accA = _sc_edge_pass(xd, g_dst, s_src, zerD)

    y1, r1 = pl.pallas_call(
        _tc_dense1_body,
        grid=(GRID,),
        in_specs=[a_spec, h_spec, n_spec, w_spec, w_spec, w_spec, w_spec],
        out_specs=[n_spec, n_spec],
        out_shape=[nd_struct, nd_struct],
    )(accA, hsrc, x, W1_l[:D], W1_l[D:], W1_r[:D], W1_r[D:])

    accB = _sc_edge_pass(y1, g_src, s_dst, zerD)

    y2, r2 = pl.pallas_call(
        _tc_dense2_body,
        grid=(GRID,),
        in_specs=[a_spec, h_spec, n_spec, b_spec, w_spec, w_spec],
        out_specs=[n_spec, n_spec],
        out_shape=[nd_struct, nd_struct],
    )(accB, hdst, r1, b1.reshape(1, D), W2_l, W2_r)

    accC = _sc_edge_pass(y2, g_src, s_dst, zerD)

    batch3 = batch.astype(jnp.int32).reshape(GRID, 1, BR)
    out = pl.pallas_call(
        _tc_pool_body,
        grid=(GRID,),
        in_specs=[a_spec, h_spec, n_spec, b_spec,
                  pl.BlockSpec((1, 1, BR), lambda i: (i, 0, 0))],
        out_specs=pl.BlockSpec((G, D), lambda i: (0, 0)),
        out_shape=jax.ShapeDtypeStruct((G, D), f32),
        scratch_shapes=[pltpu.VMEM((G, D), f32), pltpu.VMEM((G, 1), f32)],
    )(accC, hdst, r2, b2.reshape(1, D), batch3)
    return out
